# Initial kernel scaffold; baseline (speedup 1.0000x reference)
import jax
import jax.numpy as jnp
from jax.experimental import pallas as pl


def kernel(positions, lattice_values, anneal_window):
    raise NotImplementedError("write your pallas kernel here")


# speedup vs baseline: 20.2781x; 20.2781x over previous
import functools
import numpy as np
import jax
import jax.numpy as jnp
from jax import lax
from jax.experimental import pallas as pl
from jax.experimental.pallas import tpu as pltpu
from jax.experimental.pallas import tpu_sc as plsc

POS_DIM = 3
CAPACITY = 524288
NR_LEVELS = 24
NR_FEAT = 2
SCALES = (0.25 * 1.5 ** np.arange(NR_LEVELS)).astype(np.float32)
MASK = CAPACITY - 1
P1 = int(np.uint32(2654435761).astype(np.int64) - (1 << 32))
P2 = 805459861
_PS = (1 + 2654435761 + 805459861) % (1 << 32)
KPS = [int(np.int64((k * _PS) % (1 << 32)).astype(np.int32)) for k in range(4)]

NW = 32
CHUNK = 128
GROUPS = CHUNK // 16
CPT = 245
NP = NW * CPT * CHUNK


def _mesh():
    return plsc.VectorSubcoreMesh(core_axis_name="c", subcore_axis_name="s")


@functools.partial(
    pl.kernel,
    out_type=jax.ShapeDtypeStruct((NR_LEVELS * NR_FEAT, NP), jnp.float32),
    mesh=_mesh(),
    scratch_types=[
        pltpu.VMEM((POS_DIM, CHUNK), jnp.float32),
        pltpu.VMEM((NR_LEVELS, 4, 16), jnp.float32),
        pltpu.VMEM((4, CHUNK), jnp.int32),
        pltpu.VMEM((4, CHUNK), jnp.float32),
        pltpu.VMEM((4, CHUNK), jnp.float32),
        pltpu.VMEM((4, CHUNK), jnp.float32),
        pltpu.VMEM((NR_LEVELS * NR_FEAT, CHUNK), jnp.float32),
        pltpu.SemaphoreType.DMA,
    ],
)
def _permuto(pos_hbm, tab0_hbm, tab1_hbm, consts_hbm, out_hbm,
             pos_v, consts_v, idx_v, w_v, f0_v, f1_v, out_v, sem):
    wid = lax.axis_index("s") * 2 + lax.axis_index("c")
    pltpu.sync_copy(consts_hbm, consts_v)

    def chunk_body(c, carry):
        base = (wid * CPT + c) * CHUNK
        pltpu.sync_copy(pos_hbm.at[:, pl.ds(base, CHUNK)], pos_v)

        def level_body(l, carry2):
            sf0 = consts_v[l, 0]
            sf1 = consts_v[l, 1]
            sf2 = consts_v[l, 2]
            anb = consts_v[l, 3]
            lbase = l * CAPACITY
            for g in range(GROUPS):
                sl = pl.ds(g * 16, 16)
                cf0 = pos_v[0, sl] * sf0
                cf1 = pos_v[1, sl] * sf1
                cf2 = pos_v[2, sl] * sf2
                s12 = cf2 + cf1
                e = [s12 + cf0, s12 - cf0, cf2 - 2.0 * cf1, -3.0 * cf2]
                G = []
                for ei in e:
                    v = ei * 0.25
                    t = lax.convert_element_type(
                        lax.convert_element_type(v, jnp.int32), jnp.float32)
                    fl = t - jnp.where(v < t, 1.0, 0.0)
                    down = fl * 4.0
                    up = down + 4.0
                    G.append(jnp.where(up - ei < ei - down, up, down))
                _sum = lax.convert_element_type(
                    (((G[0] + G[1]) + G[2]) + G[3]) * 0.25, jnp.int32)
                diff = [e[i] - G[i] for i in range(4)]
                rank = [_sum, _sum, _sum, _sum]
                for i in range(4):
                    for j in range(i + 1, 4):
                        ci = jnp.where(diff[j] > diff[i], 1, 0)
                        rank[i] = rank[i] + ci
                        rank[j] = rank[j] + (1 - ci)
                Gi = []
                delta = []
                for i in range(4):
                    adj = (jnp.where(rank[i] < 0, 4, 0)
                           - jnp.where(rank[i] > 3, 4, 0))
                    rank[i] = rank[i] + adj
                    gf = G[i] + lax.convert_element_type(adj, jnp.float32)
                    delta.append((e[i] - gf) * 0.25)
                    if i < POS_DIM:
                        Gi.append(lax.convert_element_type(gf, jnp.int32))
                s = []
                for r in range(4):
                    acc = jnp.where(rank[0] == r, delta[0], 0.0)
                    for i in range(1, 4):
                        acc = acc + jnp.where(rank[i] == r, delta[i], 0.0)
                    s.append(acc)
                w = [s[3] + (1.0 - s[0]), s[2] - s[3], s[1] - s[2], s[0] - s[1]]
                H0 = Gi[0] + Gi[1] * P1 + Gi[2] * P2
                S = []
                for m in (1, 2, 3):
                    sm = (jnp.where(rank[0] >= m, 1, 0)
                          + jnp.where(rank[1] >= m, P1, 0)
                          + jnp.where(rank[2] >= m, P2, 0))
                    S.append(sm)
                for k in range(4):
                    h = H0 if k == 0 else H0 + KPS[k] - 4 * S[3 - k]
                    idx_v[k, sl] = (h & MASK) + lbase
                    w_v[k, sl] = w[k]
            copies = []
            for k in range(4):
                copies.append(pltpu.async_copy(
                    tab0_hbm.at[idx_v.at[k]], f0_v.at[k], sem))
                copies.append(pltpu.async_copy(
                    tab1_hbm.at[idx_v.at[k]], f1_v.at[k], sem))
            for cp in copies:
                cp.wait()
            for g in range(GROUPS):
                sl = pl.ds(g * 16, 16)
                a0 = w_v[0, sl] * f0_v[0, sl]
                a1 = w_v[0, sl] * f1_v[0, sl]
                for k in range(1, 4):
                    wk = w_v[k, sl]
                    a0 = a0 + wk * f0_v[k, sl]
                    a1 = a1 + wk * f1_v[k, sl]
                out_v[2 * l, sl] = a0 * anb
                out_v[2 * l + 1, sl] = a1 * anb
            return carry2

        lax.fori_loop(0, NR_LEVELS, level_body, 0)
        pltpu.sync_copy(out_v, out_hbm.at[:, pl.ds(base, CHUNK)])
        return carry

    lax.fori_loop(0, CPT, chunk_body, 0)


def kernel(positions, lattice_values, anneal_window):
    n = positions.shape[0]
    pos_t = jnp.pad(positions.T, ((0, 0), (0, NP - n)))
    tab = lattice_values.reshape(NR_LEVELS * CAPACITY, NR_FEAT)
    j = np.arange(1, POS_DIM + 1, dtype=np.float32)
    sf = (SCALES[:, None] / np.sqrt(j * (j + 1.0))[None, :]).astype(np.float32)
    consts = jnp.broadcast_to(
        jnp.concatenate([jnp.asarray(sf), anneal_window[:, None]],
                        axis=1)[:, :, None],
        (NR_LEVELS, 4, 16)).astype(jnp.float32)
    out = _permuto(pos_t, tab[:, 0], tab[:, 1], consts)
    return out[:, :n].T


# speedup vs baseline: 20.3344x; 1.0028x over previous
import functools
import numpy as np
import jax
import jax.numpy as jnp
from jax import lax
from jax.experimental import pallas as pl
from jax.experimental.pallas import tpu as pltpu
from jax.experimental.pallas import tpu_sc as plsc

POS_DIM = 3
CAPACITY = 524288
NR_LEVELS = 24
NR_FEAT = 2
SCALES = (0.25 * 1.5 ** np.arange(NR_LEVELS)).astype(np.float32)
MASK = CAPACITY - 1
P1 = int(np.uint32(2654435761).astype(np.int64) - (1 << 32))
P2 = 805459861
_PS = (1 + 2654435761 + 805459861) % (1 << 32)
KPS = [int(np.int64((k * _PS) % (1 << 32)).astype(np.int32)) for k in range(4)]

NW = 32
CHUNK = 128
GROUPS = CHUNK // 16
CPT = 245
NP = NW * CPT * CHUNK


def _mesh():
    return plsc.VectorSubcoreMesh(core_axis_name="c", subcore_axis_name="s")


@functools.partial(
    pl.kernel,
    out_type=jax.ShapeDtypeStruct((NR_LEVELS * NR_FEAT, NP), jnp.float32),
    mesh=_mesh(),
    scratch_types=[
        pltpu.VMEM((POS_DIM, CHUNK), jnp.float32),
        pltpu.VMEM((NR_LEVELS, 4, 16), jnp.float32),
        pltpu.VMEM((4 * CHUNK,), jnp.int32),
        pltpu.VMEM((4, CHUNK), jnp.float32),
        pltpu.VMEM((4 * CHUNK,), jnp.float32),
        pltpu.VMEM((4 * CHUNK,), jnp.float32),
        pltpu.VMEM((NR_LEVELS * NR_FEAT, CHUNK), jnp.float32),
        pltpu.SemaphoreType.DMA,
    ],
)
def _permuto(pos_hbm, tab0_hbm, tab1_hbm, consts_hbm, out_hbm,
             pos_v, consts_v, idx_v, w_v, f0_v, f1_v, out_v, sem):
    wid = lax.axis_index("s") * 2 + lax.axis_index("c")
    pltpu.sync_copy(consts_hbm, consts_v)

    def chunk_body(c, carry):
        base = (wid * CPT + c) * CHUNK
        pltpu.sync_copy(pos_hbm.at[:, pl.ds(base, CHUNK)], pos_v)

        def level_body(l, carry2):
            sf0 = consts_v[l, 0]
            sf1 = consts_v[l, 1]
            sf2 = consts_v[l, 2]
            anb = consts_v[l, 3]
            lbase = l * CAPACITY
            for g in range(GROUPS):
                sl = pl.ds(g * 16, 16)
                cf0 = pos_v[0, sl] * sf0
                cf1 = pos_v[1, sl] * sf1
                cf2 = pos_v[2, sl] * sf2
                s12 = cf2 + cf1
                e = [s12 + cf0, s12 - cf0, cf2 - 2.0 * cf1, -3.0 * cf2]
                G = []
                for ei in e:
                    v = ei * 0.25
                    t = lax.convert_element_type(
                        lax.convert_element_type(v, jnp.int32), jnp.float32)
                    fl = t - jnp.where(v < t, 1.0, 0.0)
                    down = fl * 4.0
                    up = down + 4.0
                    G.append(jnp.where(up - ei < ei - down, up, down))
                _sum = lax.convert_element_type(
                    (((G[0] + G[1]) + G[2]) + G[3]) * 0.25, jnp.int32)
                diff = [e[i] - G[i] for i in range(4)]
                rank = [_sum, _sum, _sum, _sum]
                for i in range(4):
                    for j in range(i + 1, 4):
                        ci = jnp.where(diff[j] > diff[i], 1, 0)
                        rank[i] = rank[i] + ci
                        rank[j] = rank[j] + (1 - ci)
                Gi = []
                delta = []
                for i in range(4):
                    adj = (jnp.where(rank[i] < 0, 4, 0)
                           - jnp.where(rank[i] > 3, 4, 0))
                    rank[i] = rank[i] + adj
                    gf = G[i] + lax.convert_element_type(adj, jnp.float32)
                    delta.append((e[i] - gf) * 0.25)
                    if i < POS_DIM:
                        Gi.append(lax.convert_element_type(gf, jnp.int32))
                s = []
                for r in range(4):
                    acc = jnp.where(rank[0] == r, delta[0], 0.0)
                    for i in range(1, 4):
                        acc = acc + jnp.where(rank[i] == r, delta[i], 0.0)
                    s.append(acc)
                w = [s[3] + (1.0 - s[0]), s[2] - s[3], s[1] - s[2], s[0] - s[1]]
                H0 = Gi[0] + Gi[1] * P1 + Gi[2] * P2
                S = []
                for m in (1, 2, 3):
                    sm = (jnp.where(rank[0] >= m, 1, 0)
                          + jnp.where(rank[1] >= m, P1, 0)
                          + jnp.where(rank[2] >= m, P2, 0))
                    S.append(sm)
                for k in range(4):
                    h = H0 if k == 0 else H0 + KPS[k] - 4 * S[3 - k]
                    idx_v[pl.ds(k * CHUNK + g * 16, 16)] = (h & MASK) + lbase
                    w_v[k, sl] = w[k]
            cp0 = pltpu.async_copy(tab0_hbm.at[idx_v], f0_v, sem)
            cp1 = pltpu.async_copy(tab1_hbm.at[idx_v], f1_v, sem)
            cp0.wait()
            cp1.wait()
            for g in range(GROUPS):
                sl = pl.ds(g * 16, 16)
                a0 = w_v[0, sl] * f0_v[pl.ds(g * 16, 16)]
                a1 = w_v[0, sl] * f1_v[pl.ds(g * 16, 16)]
                for k in range(1, 4):
                    wk = w_v[k, sl]
                    slk = pl.ds(k * CHUNK + g * 16, 16)
                    a0 = a0 + wk * f0_v[slk]
                    a1 = a1 + wk * f1_v[slk]
                out_v[2 * l, sl] = a0 * anb
                out_v[2 * l + 1, sl] = a1 * anb
            return carry2

        lax.fori_loop(0, NR_LEVELS, level_body, 0)
        pltpu.sync_copy(out_v, out_hbm.at[:, pl.ds(base, CHUNK)])
        return carry

    lax.fori_loop(0, CPT, chunk_body, 0)


def kernel(positions, lattice_values, anneal_window):
    n = positions.shape[0]
    pos_t = jnp.pad(positions.T, ((0, 0), (0, NP - n)))
    tab = lattice_values.reshape(NR_LEVELS * CAPACITY, NR_FEAT)
    j = np.arange(1, POS_DIM + 1, dtype=np.float32)
    sf = (SCALES[:, None] / np.sqrt(j * (j + 1.0))[None, :]).astype(np.float32)
    consts = jnp.broadcast_to(
        jnp.concatenate([jnp.asarray(sf), anneal_window[:, None]],
                        axis=1)[:, :, None],
        (NR_LEVELS, 4, 16)).astype(jnp.float32)
    out = _permuto(pos_t, tab[:, 0], tab[:, 1], consts)
    return out[:, :n].T


# speedup vs baseline: 21.5790x; 1.0612x over previous
import functools
import numpy as np
import jax
import jax.numpy as jnp
from jax import lax
from jax.experimental import pallas as pl
from jax.experimental.pallas import tpu as pltpu
from jax.experimental.pallas import tpu_sc as plsc

POS_DIM = 3
CAPACITY = 524288
NR_LEVELS = 24
NR_FEAT = 2
SCALES = (0.25 * 1.5 ** np.arange(NR_LEVELS)).astype(np.float32)
MASK = CAPACITY - 1
P1 = int(np.uint32(2654435761).astype(np.int64) - (1 << 32))
P2 = 805459861
_PS = (1 + 2654435761 + 805459861) % (1 << 32)
KPS = [int(np.int64((k * _PS) % (1 << 32)).astype(np.int32)) for k in range(4)]

NW = 32
CHUNK = 128
GROUPS = CHUNK // 16
CPT = 245
NP = NW * CPT * CHUNK


def _mesh():
    return plsc.VectorSubcoreMesh(core_axis_name="c", subcore_axis_name="s")


@functools.partial(
    pl.kernel,
    out_type=jax.ShapeDtypeStruct((NR_LEVELS * NR_FEAT, NP), jnp.float32),
    mesh=_mesh(),
    scratch_types=[
        pltpu.VMEM((POS_DIM, CHUNK), jnp.float32),
        pltpu.VMEM((NR_LEVELS, 4, 16), jnp.float32),
        pltpu.VMEM((4 * CHUNK,), jnp.int32),
        pltpu.VMEM((4 * CHUNK,), jnp.int32),
        pltpu.VMEM((4, CHUNK), jnp.float32),
        pltpu.VMEM((4, CHUNK), jnp.float32),
        pltpu.VMEM((4 * CHUNK,), jnp.float32),
        pltpu.VMEM((4 * CHUNK,), jnp.float32),
        pltpu.VMEM((4 * CHUNK,), jnp.float32),
        pltpu.VMEM((4 * CHUNK,), jnp.float32),
        pltpu.VMEM((NR_LEVELS * NR_FEAT, CHUNK), jnp.float32),
        pltpu.SemaphoreType.DMA,
    ],
)
def _permuto(pos_hbm, tab0_hbm, tab1_hbm, consts_hbm, out_hbm,
             pos_v, consts_v, idx_a, idx_b, w_a, w_b,
             f0_a, f0_b, f1_a, f1_b, out_v, sem):
    wid = lax.axis_index("s") * 2 + lax.axis_index("c")
    pltpu.sync_copy(consts_hbm, consts_v)
    bufs = ((idx_a, w_a, f0_a, f1_a), (idx_b, w_b, f0_b, f1_b))

    def compute_and_fire(l, buf):
        idx_v, w_v, f0_v, f1_v = buf
        sf0 = consts_v[l, 0]
        sf1 = consts_v[l, 1]
        sf2 = consts_v[l, 2]
        lbase = l * CAPACITY
        for g in range(GROUPS):
            sl = pl.ds(g * 16, 16)
            cf0 = pos_v[0, sl] * sf0
            cf1 = pos_v[1, sl] * sf1
            cf2 = pos_v[2, sl] * sf2
            s12 = cf2 + cf1
            e = [s12 + cf0, s12 - cf0, cf2 - 2.0 * cf1, -3.0 * cf2]
            G = []
            for ei in e:
                v = ei * 0.25
                t = lax.convert_element_type(
                    lax.convert_element_type(v, jnp.int32), jnp.float32)
                fl = t - jnp.where(v < t, 1.0, 0.0)
                down = fl * 4.0
                up = down + 4.0
                G.append(jnp.where(up - ei < ei - down, up, down))
            _sum = lax.convert_element_type(
                (((G[0] + G[1]) + G[2]) + G[3]) * 0.25, jnp.int32)
            diff = [e[i] - G[i] for i in range(4)]
            rank = [_sum, _sum, _sum, _sum]
            for i in range(4):
                for j in range(i + 1, 4):
                    ci = jnp.where(diff[j] > diff[i], 1, 0)
                    rank[i] = rank[i] + ci
                    rank[j] = rank[j] + (1 - ci)
            Gi = []
            delta = []
            for i in range(4):
                adj = (jnp.where(rank[i] < 0, 4, 0)
                       - jnp.where(rank[i] > 3, 4, 0))
                rank[i] = rank[i] + adj
                gf = G[i] + lax.convert_element_type(adj, jnp.float32)
                delta.append((e[i] - gf) * 0.25)
                if i < POS_DIM:
                    Gi.append(lax.convert_element_type(gf, jnp.int32))
            s = []
            for r in range(4):
                acc = jnp.where(rank[0] == r, delta[0], 0.0)
                for i in range(1, 4):
                    acc = acc + jnp.where(rank[i] == r, delta[i], 0.0)
                s.append(acc)
            w = [s[3] + (1.0 - s[0]), s[2] - s[3], s[1] - s[2], s[0] - s[1]]
            H0 = Gi[0] + Gi[1] * P1 + Gi[2] * P2
            S = []
            for m in (1, 2, 3):
                sm = (jnp.where(rank[0] >= m, 1, 0)
                      + jnp.where(rank[1] >= m, P1, 0)
                      + jnp.where(rank[2] >= m, P2, 0))
                S.append(sm)
            for k in range(4):
                h = H0 if k == 0 else H0 + KPS[k] - 4 * S[3 - k]
                idx_v[pl.ds(k * CHUNK + g * 16, 16)] = (h & MASK) + lbase
                w_v[k, sl] = w[k]
        pltpu.async_copy(tab0_hbm.at[idx_v], f0_v, sem)
        pltpu.async_copy(tab1_hbm.at[idx_v], f1_v, sem)

    def wait_and_accumulate(l, buf):
        idx_v, w_v, f0_v, f1_v = buf
        pltpu.make_async_copy(tab0_hbm.at[idx_v], f0_v, sem).wait()
        pltpu.make_async_copy(tab1_hbm.at[idx_v], f1_v, sem).wait()
        anb = consts_v[l, 3]
        for g in range(GROUPS):
            sl = pl.ds(g * 16, 16)
            a0 = w_v[0, sl] * f0_v[pl.ds(g * 16, 16)]
            a1 = w_v[0, sl] * f1_v[pl.ds(g * 16, 16)]
            for k in range(1, 4):
                wk = w_v[k, sl]
                slk = pl.ds(k * CHUNK + g * 16, 16)
                a0 = a0 + wk * f0_v[slk]
                a1 = a1 + wk * f1_v[slk]
            out_v[2 * l, sl] = a0 * anb
            out_v[2 * l + 1, sl] = a1 * anb

    def chunk_body(c, carry):
        base = (wid * CPT + c) * CHUNK
        pltpu.sync_copy(pos_hbm.at[:, pl.ds(base, CHUNK)], pos_v)

        def level_body(l, carry2):
            b = lax.rem(l, 2)
            @pl.when(jnp.logical_and(l < NR_LEVELS, b == 0))
            def _():
                compute_and_fire(l, bufs[0])
            @pl.when(jnp.logical_and(l < NR_LEVELS, b == 1))
            def _():
                compute_and_fire(l, bufs[1])
            @pl.when(jnp.logical_and(l > 0, b == 1))
            def _():
                wait_and_accumulate(l - 1, bufs[0])
            @pl.when(jnp.logical_and(l > 0, b == 0))
            def _():
                wait_and_accumulate(l - 1, bufs[1])
            return carry2

        lax.fori_loop(0, NR_LEVELS + 1, level_body, 0)
        pltpu.sync_copy(out_v, out_hbm.at[:, pl.ds(base, CHUNK)])
        return carry

    lax.fori_loop(0, CPT, chunk_body, 0)


def kernel(positions, lattice_values, anneal_window):
    n = positions.shape[0]
    pos_t = jnp.pad(positions.T, ((0, 0), (0, NP - n)))
    tab = lattice_values.reshape(NR_LEVELS * CAPACITY, NR_FEAT)
    j = np.arange(1, POS_DIM + 1, dtype=np.float32)
    sf = (SCALES[:, None] / np.sqrt(j * (j + 1.0))[None, :]).astype(np.float32)
    consts = jnp.broadcast_to(
        jnp.concatenate([jnp.asarray(sf), anneal_window[:, None]],
                        axis=1)[:, :, None],
        (NR_LEVELS, 4, 16)).astype(jnp.float32)
    out = _permuto(pos_t, tab[:, 0], tab[:, 1], consts)
    return out[:, :n].T
